# baseline (speedup 1.0000x reference)
import functools

import jax
import jax.numpy as jnp
from jax import lax
from jax.experimental import pallas as pl
from jax.experimental.pallas import tpu as pltpu
from jax.experimental.pallas import tpu_sc as plsc

_DIMS = 32
_PACK = 4
_NUM_WORKERS = 32


def _embedding_gather(batch, hist):
    b_per_w = batch // _NUM_WORKERS
    half = b_per_w // 2
    n_bg = half // 16
    mesh = plsc.VectorSubcoreMesh(core_axis_name="c", subcore_axis_name="s")

    scratch = (
        [pltpu.VMEM((hist, b_per_w), jnp.int32) for _ in range(3)]
        + [pltpu.VMEM((half, _PACK * _DIMS), jnp.float32) for _ in range(2)]
        + [pltpu.VMEM((_DIMS, half), jnp.float32) for _ in range(2)]
        + [pltpu.SemaphoreType.DMA for _ in range(5)]
    )

    @functools.partial(
        pl.kernel,
        mesh=mesh,
        out_type=jax.ShapeDtypeStruct((hist, _DIMS, batch), jnp.float32),
        scratch_types=scratch,
        compiler_params=pltpu.CompilerParams(
            use_tc_tiling_on_sc=False, needs_layout_passes=False
        ),
    )
    def k(idx_hbm, table_hbm, out_hbm, idx_v, q_v, co_v,
          r0, r1, t0, t1, g0, g1, o0, o1, s_idx):
        rows = (r0, r1)
        outb = (t0, t1)
        g_sem = (g0, g1)
        o_sem = (o0, o1)

        wid = lax.axis_index("s") * 2 + lax.axis_index("c")
        b0 = wid * b_per_w
        lane = jnp.arange(16, dtype=jnp.int32)

        pltpu.sync_copy(idx_hbm.at[:, pl.ds(b0, b_per_w)], idx_v)

        def precomp(i, carry):
            h = i // (b_per_w // 16)
            bg = i % (b_per_w // 16)
            v = idx_v[h, pl.ds(bg * 16, 16)]
            q_v[h, pl.ds(bg * 16, 16)] = lax.shift_right_logical(v, 2)
            co_v[h, pl.ds(bg * 16, 16)] = lax.shift_left(v & 3, 5)
            return carry

        lax.fori_loop(0, hist * (b_per_w // 16), precomp, 0)

        def start_gather(h, hf):
            b = hf % 2
            return pltpu.async_copy(
                table_hbm.at[q_v.at[h, pl.ds(hf * half, half)]], rows[b], g_sem[b]
            )

        def start_out(h, hf):
            b = hf % 2
            return pltpu.async_copy(
                outb[b], out_hbm.at[h, :, pl.ds(b0 + hf * half, half)], o_sem[b]
            )

        def wait_out(h, hf):
            b = hf % 2
            pltpu.make_async_copy(
                outb[b], out_hbm.at[h, :, pl.ds(b0 + hf * half, half)], o_sem[b]
            ).wait()

        def wait_gather(h, hf):
            b = hf % 2
            pltpu.make_async_copy(
                table_hbm.at[q_v.at[h, pl.ds(hf * half, half)]], rows[b], g_sem[b]
            ).wait()

        def transpose(h, hf):
            b = hf % 2
            src = rows[b]
            dst = outb[b]

            def body(bg, carry):
                bids = bg * 16 + lane
                co = co_v[h, pl.ds(hf * half + bg * 16, 16)]
                for d0 in range(_DIMS):
                    dims = (d0 + lane) & (_DIMS - 1)
                    v = plsc.load_gather(src, [bids, co + dims])
                    plsc.store_scatter(dst, [dims, bids], v)
                return carry

            lax.fori_loop(0, n_bg, body, 0)

        gathers = {}
        start_gather(0, 0)

        def step(h, carry):
            start_gather(h, 1)
            wait_gather(h, 0)

            @pl.when(h > 0)
            def _():
                wait_out(h - 1, 0)

            transpose(h, 0)
            start_out(h, 0)

            @pl.when(h + 1 < hist)
            def _():
                start_gather(h + 1, 0)

            wait_gather(h, 1)

            @pl.when(h > 0)
            def _():
                wait_out(h - 1, 1)

            transpose(h, 1)
            start_out(h, 1)
            return carry

        lax.fori_loop(0, hist, step, 0)
        wait_out(hist - 1, 0)
        wait_out(hist - 1, 1)

    return k


def kernel(inputs, weight):
    batch, hist = inputs.shape
    num_rows, dims = weight.shape
    wt_packed = weight.reshape(num_rows // _PACK, _PACK * dims)
    out_t = _embedding_gather(batch, hist)(inputs.T, wt_packed)
    return out_t.transpose(2, 0, 1)


# speedup vs baseline: 1.0920x; 1.0920x over previous
import functools

import jax
import jax.numpy as jnp
from jax import lax
from jax.experimental import pallas as pl
from jax.experimental.pallas import tpu as pltpu
from jax.experimental.pallas import tpu_sc as plsc

_DIMS = 32
_PACK = 4
_NUM_WORKERS = 32


def _embedding_gather(batch, hist):
    b_per_w = batch // _NUM_WORKERS
    half = b_per_w // 2
    n_bg = half // 16
    mesh = plsc.VectorSubcoreMesh(core_axis_name="c", subcore_axis_name="s")

    scratch = (
        [pltpu.VMEM((hist, b_per_w), jnp.int32)]
        + [pltpu.VMEM((hist * b_per_w,), jnp.int32)]
        + [pltpu.VMEM((hist, b_per_w), jnp.int32)]
        + [pltpu.VMEM((half, _PACK * _DIMS), jnp.float32) for _ in range(2)]
        + [pltpu.VMEM((_DIMS, half), jnp.float32) for _ in range(2)]
        + [pltpu.SemaphoreType.DMA for _ in range(5)]
    )

    @functools.partial(
        pl.kernel,
        mesh=mesh,
        out_type=jax.ShapeDtypeStruct((hist, _DIMS, batch), jnp.float32),
        scratch_types=scratch,
        compiler_params=pltpu.CompilerParams(
            use_tc_tiling_on_sc=True, needs_layout_passes=False
        ),
    )
    def k(idx_hbm, table_hbm, out_hbm, idx_v, q_v, co_v,
          r0, r1, t0, t1, g0, g1, o0, o1, s_idx):
        rows = (r0, r1)
        outb = (t0, t1)
        g_sem = (g0, g1)
        o_sem = (o0, o1)

        wid = lax.axis_index("s") * 2 + lax.axis_index("c")
        b0 = wid * b_per_w
        lane = jnp.arange(16, dtype=jnp.int32)

        pltpu.sync_copy(idx_hbm.at[:, pl.ds(b0, b_per_w)], idx_v)

        def precomp(i, carry):
            h = i // (b_per_w // 16)
            bg = i % (b_per_w // 16)
            v = idx_v[h, pl.ds(bg * 16, 16)]
            q_v[pl.ds(i * 16, 16)] = lax.shift_right_logical(v, 2)
            co_v[h, pl.ds(bg * 16, 16)] = lax.shift_left(v & 3, 5)
            return carry

        lax.fori_loop(0, hist * (b_per_w // 16), precomp, 0)

        def start_gather(h, hf):
            b = hf % 2
            return pltpu.async_copy(
                table_hbm.at[q_v.at[pl.ds(h * b_per_w + hf * half, half)]], rows[b], g_sem[b]
            )

        def start_out(h, hf):
            b = hf % 2
            return pltpu.async_copy(
                outb[b], out_hbm.at[h, :, pl.ds(b0 + hf * half, half)], o_sem[b]
            )

        def wait_out(h, hf):
            b = hf % 2
            pltpu.make_async_copy(
                outb[b], out_hbm.at[h, :, pl.ds(b0 + hf * half, half)], o_sem[b]
            ).wait()

        def wait_gather(h, hf):
            b = hf % 2
            pltpu.make_async_copy(
                table_hbm.at[q_v.at[pl.ds(h * b_per_w + hf * half, half)]], rows[b], g_sem[b]
            ).wait()

        def transpose(h, hf):
            b = hf % 2
            src = rows[b]
            dst = outb[b]

            def body(bg, carry):
                bids = bg * 16 + lane
                co = co_v[h, pl.ds(hf * half + bg * 16, 16)]
                for d0 in range(_DIMS):
                    dims = (d0 + lane) & (_DIMS - 1)
                    v = plsc.load_gather(src, [bids, co + dims])
                    plsc.store_scatter(dst, [dims, bids], v)
                return carry

            lax.fori_loop(0, n_bg, body, 0)

        gathers = {}
        start_gather(0, 0)

        def step(h, carry):
            start_gather(h, 1)
            wait_gather(h, 0)

            @pl.when(h > 0)
            def _():
                wait_out(h - 1, 0)

            transpose(h, 0)
            start_out(h, 0)

            @pl.when(h + 1 < hist)
            def _():
                start_gather(h + 1, 0)

            wait_gather(h, 1)

            @pl.when(h > 0)
            def _():
                wait_out(h - 1, 1)

            transpose(h, 1)
            start_out(h, 1)
            return carry

        lax.fori_loop(0, hist, step, 0)
        wait_out(hist - 1, 0)
        wait_out(hist - 1, 1)

    return k


def kernel(inputs, weight):
    batch, hist = inputs.shape
    num_rows, dims = weight.shape
    wt_packed = weight.reshape(num_rows // _PACK, _PACK * dims)
    out_t = _embedding_gather(batch, hist)(inputs.T, wt_packed)
    return out_t.transpose(2, 0, 1)
